# Initial kernel scaffold; baseline (speedup 1.0000x reference)
import jax
import jax.numpy as jnp
from jax.experimental import pallas as pl


def kernel(x, edge_index, W1, b1, ln_g, ln_b, Wg1, bg1, Wg2, bg2, Wg3, bg3, memory, Wq, bq, Wk, bk, Wv, bv, Wo, bo, Wc1, bc1, Wc2, bc2):
    raise NotImplementedError("write your pallas kernel here")


# speedup vs baseline: 10.1535x; 10.1535x over previous
import functools

import jax
import jax.numpy as jnp
from jax import lax
from jax.experimental import pallas as pl
from jax.experimental.pallas import tpu as pltpu
from jax.experimental.pallas import tpu_sc as plsc

N = 10000
D_IN = 128
H = 128
NUM_CLASSES = 10
MEM = 64
NH = 8
HD = H // NH

NC = 2
NS = 16
NW = NC * NS
CHUNK = 128
ROWS_PER_TILE = 640
NROWS = NS * ROWS_PER_TILE
ZR = 64

BR = 1000


def _sc_mesh():
    return plsc.VectorSubcoreMesh(
        core_axis_name="c", subcore_axis_name="s", num_cores=NC, num_subcores=NS)


def _make_sc_deg(ncu):
    def body(dstp_hbm, zeros_hbm, ones_hbm, out_hbm, dst_v, ones_v, zbuf, acc, sem):
        c = lax.axis_index("c")
        s = lax.axis_index("s")
        wid = s * NC + c
        base = s * ROWS_PER_TILE
        pltpu.sync_copy(zeros_hbm, zbuf)
        for r in range(ROWS_PER_TILE // ZR):
            pltpu.sync_copy(zbuf, acc.at[pl.ds(base + r * ZR, ZR)])
        pltpu.sync_copy(ones_hbm, ones_v)
        pltpu.sync_copy(dstp_hbm.at[wid], dst_v)
        plsc.subcore_barrier()

        def step(j, carry):
            pltpu.sync_copy(ones_v, acc.at[dst_v.at[j]], add=True)
            return carry

        lax.fori_loop(0, ncu, step, 0)
        plsc.subcore_barrier()
        for r in range(ROWS_PER_TILE // ZR):
            pltpu.sync_copy(acc.at[pl.ds(base + r * ZR, ZR)], zbuf)
            pltpu.sync_copy(zbuf, out_hbm.at[c, pl.ds(base + r * ZR, ZR)])

    return pl.kernel(
        body,
        out_type=jax.ShapeDtypeStruct((NC, NROWS, H), jnp.float32),
        mesh=_sc_mesh(),
        scratch_types=[
            pltpu.VMEM((ncu, CHUNK), jnp.int32),
            pltpu.VMEM((CHUNK, H), jnp.float32),
            pltpu.VMEM((ZR, H), jnp.float32),
            pltpu.VMEM_SHARED((NROWS, H), jnp.float32),
            pltpu.SemaphoreType.DMA,
        ],
    )


def _make_sc_scatter(ncu):
    def body(y_hbm, srcp_hbm, dstp_hbm, zeros_hbm, out_hbm,
             src_v, dst_v, rows_v, zbuf, acc, sem):
        c = lax.axis_index("c")
        s = lax.axis_index("s")
        wid = s * NC + c
        base = s * ROWS_PER_TILE
        pltpu.sync_copy(zeros_hbm, zbuf)
        for r in range(ROWS_PER_TILE // ZR):
            pltpu.sync_copy(zbuf, acc.at[pl.ds(base + r * ZR, ZR)])
        pltpu.sync_copy(srcp_hbm.at[wid], src_v)
        pltpu.sync_copy(dstp_hbm.at[wid], dst_v)
        plsc.subcore_barrier()

        def step(j, carry):
            pltpu.async_copy(y_hbm.at[src_v.at[j]], rows_v, sem).wait()
            pltpu.sync_copy(rows_v, acc.at[dst_v.at[j]], add=True)
            return carry

        lax.fori_loop(0, ncu, step, 0)
        plsc.subcore_barrier()
        for r in range(ROWS_PER_TILE // ZR):
            pltpu.sync_copy(acc.at[pl.ds(base + r * ZR, ZR)], zbuf)
            pltpu.sync_copy(zbuf, out_hbm.at[c, pl.ds(base + r * ZR, ZR)])

    return pl.kernel(
        body,
        out_type=jax.ShapeDtypeStruct((NC, NROWS, H), jnp.float32),
        mesh=_sc_mesh(),
        scratch_types=[
            pltpu.VMEM((ncu, CHUNK), jnp.int32),
            pltpu.VMEM((ncu, CHUNK), jnp.int32),
            pltpu.VMEM((CHUNK, H), jnp.float32),
            pltpu.VMEM((ZR, H), jnp.float32),
            pltpu.VMEM_SHARED((NROWS, H), jnp.float32),
            pltpu.SemaphoreType.DMA,
        ],
    )


def _dis_from(deg_ref):
    deg = deg_ref[0, :, 0:1] + deg_ref[1, :, 0:1] + 1.0
    return lax.rsqrt(deg)


def _tc_a_body(x_ref, w1_ref, b1_ref, g_ref, bln_ref, wg_ref, deg_ref, y_ref):
    t = jnp.dot(x_ref[...], w1_ref[...], preferred_element_type=jnp.float32)
    t = jnp.maximum(t + b1_ref[...], 0.0)
    mu = jnp.mean(t, axis=-1, keepdims=True)
    var = jnp.mean((t - mu) ** 2, axis=-1, keepdims=True)
    t = (t - mu) * lax.rsqrt(var + 1e-5) * g_ref[...] + bln_ref[...]
    dis = _dis_from(deg_ref)
    y_ref[...] = dis * jnp.dot(t, wg_ref[...], preferred_element_type=jnp.float32)


def _tc_b_body(s_ref, y_ref, deg_ref, b_ref, w_ref, o_ref):
    dis = _dis_from(deg_ref)
    h = jnp.maximum(b_ref[...] + dis * (s_ref[0] + s_ref[1] + y_ref[...]), 0.0)
    o_ref[...] = dis * jnp.dot(h, w_ref[...], preferred_element_type=jnp.float32)


def _tc_c1_body(s_ref, y_ref, deg_ref, b_ref, o_ref):
    i = pl.program_id(0)
    dis = _dis_from(deg_ref)
    h = jnp.maximum(b_ref[...] + dis * (s_ref[0] + s_ref[1] + y_ref[...]), 0.0)
    ps = jnp.sum(h, axis=0, keepdims=True)

    @pl.when(i == 0)
    def _():
        o_ref[...] = ps

    @pl.when(i != 0)
    def _():
        o_ref[...] += ps


def _tc_c2_body(ps_ref, mem_ref, wq_ref, bq_ref, wk_ref, bk_ref, wv_ref, bv_ref,
                wo_ref, bo_ref, wc1_ref, bc1_ref, wc2_ref, bc2_ref, o_ref):
    pooled = ps_ref[...] * (1.0 / N)
    q = jnp.dot(pooled, wq_ref[...], preferred_element_type=jnp.float32) + bq_ref[...]
    k = jnp.dot(mem_ref[...], wk_ref[...], preferred_element_type=jnp.float32) + bk_ref[...]
    v = jnp.dot(mem_ref[...], wv_ref[...], preferred_element_type=jnp.float32) + bv_ref[...]
    m = (lax.broadcasted_iota(jnp.int32, (H, NH), 0) // HD
         == lax.broadcasted_iota(jnp.int32, (H, NH), 1)).astype(jnp.float32)
    mt = (lax.broadcasted_iota(jnp.int32, (NH, H), 1) // HD
          == lax.broadcasted_iota(jnp.int32, (NH, H), 0)).astype(jnp.float32)
    scores = jnp.dot(k * q, m, preferred_element_type=jnp.float32) * (HD ** -0.5)
    mx = jnp.max(scores, axis=0, keepdims=True)
    e = jnp.exp(scores - mx)
    attn = e / jnp.sum(e, axis=0, keepdims=True)
    attn_exp = jnp.dot(attn, mt, preferred_element_type=jnp.float32)
    o = jnp.sum(attn_exp * v, axis=0, keepdims=True)
    attended = jnp.dot(o, wo_ref[...], preferred_element_type=jnp.float32) + bo_ref[...]
    hidden = jnp.maximum(
        jnp.dot(pooled, wc1_ref[0:H], preferred_element_type=jnp.float32)
        + jnp.dot(attended, wc1_ref[H:2 * H], preferred_element_type=jnp.float32)
        + bc1_ref[...], 0.0)
    o_ref[...] = jnp.dot(hidden, wc2_ref[...], preferred_element_type=jnp.float32) + bc2_ref[...]


def _row_block(i):
    return (i, 0)


def _tc_a(x, w1, b1, g, bln, wg, degp):
    grid = (N // BR,)
    return pl.pallas_call(
        _tc_a_body,
        grid=grid,
        in_specs=[
            pl.BlockSpec((BR, D_IN), _row_block),
            pl.BlockSpec((D_IN, H), lambda i: (0, 0)),
            pl.BlockSpec((1, H), lambda i: (0, 0)),
            pl.BlockSpec((1, H), lambda i: (0, 0)),
            pl.BlockSpec((1, H), lambda i: (0, 0)),
            pl.BlockSpec((H, H), lambda i: (0, 0)),
            pl.BlockSpec((NC, BR, H), lambda i: (0, i, 0)),
        ],
        out_specs=pl.BlockSpec((BR, H), _row_block),
        out_shape=jax.ShapeDtypeStruct((N, H), jnp.float32),
    )(x, w1, b1, g, bln, wg, degp)


def _tc_b(s2, y, degp, b, w):
    grid = (N // BR,)
    return pl.pallas_call(
        _tc_b_body,
        grid=grid,
        in_specs=[
            pl.BlockSpec((NC, BR, H), lambda i: (0, i, 0)),
            pl.BlockSpec((BR, H), _row_block),
            pl.BlockSpec((NC, BR, H), lambda i: (0, i, 0)),
            pl.BlockSpec((1, H), lambda i: (0, 0)),
            pl.BlockSpec((H, H), lambda i: (0, 0)),
        ],
        out_specs=pl.BlockSpec((BR, H), _row_block),
        out_shape=jax.ShapeDtypeStruct((N, H), jnp.float32),
    )(s2, y, degp, b, w)


def _tc_c1(s2, y, degp, b):
    grid = (N // BR,)
    return pl.pallas_call(
        _tc_c1_body,
        grid=grid,
        in_specs=[
            pl.BlockSpec((NC, BR, H), lambda i: (0, i, 0)),
            pl.BlockSpec((BR, H), _row_block),
            pl.BlockSpec((NC, BR, H), lambda i: (0, i, 0)),
            pl.BlockSpec((1, H), lambda i: (0, 0)),
        ],
        out_specs=pl.BlockSpec((1, H), lambda i: (0, 0)),
        out_shape=jax.ShapeDtypeStruct((1, H), jnp.float32),
    )(s2, y, degp, b)


def _tc_c2(ps, memory, wq, bq, wk, bk, wv, bv, wo, bo, wc1, bc1, wc2, bc2):
    return pl.pallas_call(
        _tc_c2_body,
        out_shape=jax.ShapeDtypeStruct((1, NUM_CLASSES), jnp.float32),
    )(ps, memory, wq, bq, wk, bk, wv, bv, wo, bo, wc1, bc1, wc2, bc2)


def kernel(x, edge_index, W1, b1, ln_g, ln_b, Wg1, bg1, Wg2, bg2, Wg3, bg3,
           memory, Wq, bq, Wk, bk, Wv, bv, Wo, bo, Wc1, bc1, Wc2, bc2):
    src = edge_index[0]
    dst = edge_index[1]
    e = src.shape[0]
    epg = NW * CHUNK
    ncu = -(-e // epg)
    pad = ncu * epg - e
    srcp = jnp.concatenate([src, jnp.zeros((pad,), jnp.int32)]).reshape(NW, ncu, CHUNK)
    dstp = jnp.concatenate([dst, jnp.full((pad,), N, jnp.int32)]).reshape(NW, ncu, CHUNK)
    zeros_h = jnp.zeros((ZR, H), jnp.float32)
    ones_h = jnp.ones((CHUNK, H), jnp.float32)

    b1r = b1.reshape(1, H)
    gr = ln_g.reshape(1, H)
    blnr = ln_b.reshape(1, H)
    bg1r = bg1.reshape(1, H)
    bg2r = bg2.reshape(1, H)
    bg3r = bg3.reshape(1, H)

    sc_deg = _make_sc_deg(ncu)
    sc_scatter = _make_sc_scatter(ncu)

    degp = sc_deg(dstp, zeros_h, ones_h)
    y1 = _tc_a(x, W1, b1r, gr, blnr, Wg1, degp)
    s1 = sc_scatter(y1, srcp, dstp, zeros_h)
    y2 = _tc_b(s1, y1, degp, bg1r, Wg2)
    s2 = sc_scatter(y2, srcp, dstp, zeros_h)
    y3 = _tc_b(s2, y2, degp, bg2r, Wg3)
    s3 = sc_scatter(y3, srcp, dstp, zeros_h)
    ps = _tc_c1(s3, y3, degp, bg3r)
    return _tc_c2(ps, memory, Wq.reshape(H, H), bq.reshape(1, H), Wk, bk.reshape(1, H),
                  Wv, bv.reshape(1, H), Wo, bo.reshape(1, H),
                  Wc1, bc1.reshape(1, H), Wc2, bc2.reshape(1, NUM_CLASSES))
